# baseline (speedup 1.0000x reference)
import functools

import jax
import jax.numpy as jnp
from jax import lax
from jax.experimental import pallas as pl
from jax.experimental.pallas import tpu as pltpu
from jax.experimental.pallas import tpu_sc as plsc

CUTOFF = 5.0
_CUT2_EDGE = 24.999998092651367
NC, NS, L = 2, 16, 16
NW = NC * NS

_COS_C = (
    0.999998250310558,
    -4.934728306205929,
    4.057967848533368,
    -1.3322538414683185,
    0.22958862108713401,
    -0.020574270313145616,
)
_SW_C = tuple(0.5 * c for c in _COS_C)
_SW_C = (_SW_C[0] + 0.5,) + _SW_C[1:]


def _make_kernel(n_nodes: int, n_edges: int, chunk: int):
    assert n_edges % (NW * chunk) == 0
    e_per_w = n_edges // NW
    n_chunks = e_per_w // chunk
    assert n_chunks % 2 == 0 and chunk % L == 0
    mesh = plsc.VectorSubcoreMesh(core_axis_name="c", subcore_axis_name="s")
    inv_cut2 = 1.0 / (CUTOFF * CUTOFF)

    @functools.partial(
        pl.kernel,
        mesh=mesh,
        out_type=[
            jax.ShapeDtypeStruct((n_edges,), jnp.float32),
            jax.ShapeDtypeStruct((n_edges,), jnp.float32),
            jax.ShapeDtypeStruct((n_edges,), jnp.float32),
            jax.ShapeDtypeStruct((n_edges,), jnp.float32),
            jax.ShapeDtypeStruct((n_edges,), jnp.float32),
            jax.ShapeDtypeStruct((n_edges,), jnp.int32),
        ],
        scratch_types=[
            pltpu.VMEM((n_nodes,), jnp.float32),
            pltpu.VMEM((chunk,), jnp.int32),
            pltpu.VMEM((chunk,), jnp.int32),
            pltpu.VMEM((chunk,), jnp.int32),
            pltpu.VMEM((chunk,), jnp.int32),
            pltpu.VMEM((chunk,), jnp.float32),
            pltpu.VMEM((chunk,), jnp.float32),
            pltpu.VMEM((chunk,), jnp.float32),
            pltpu.VMEM((chunk,), jnp.float32),
            pltpu.VMEM((chunk,), jnp.float32),
            pltpu.VMEM((chunk,), jnp.float32),
            pltpu.VMEM((chunk,), jnp.int32),
            pltpu.VMEM((chunk,), jnp.float32),
            pltpu.SemaphoreType.DMA,
            pltpu.SemaphoreType.DMA,
            pltpu.SemaphoreType.DMA,
            pltpu.SemaphoreType.DMA,
            pltpu.SemaphoreType.DMA,
        ],
        compiler_params=pltpu.CompilerParams(needs_layout_passes=False),
    )
    def k(xs_hbm, ys_hbm, zs_hbm, src_hbm, dst_hbm,
          vx_hbm, vy_hbm, vz_hbm, dist_hbm, sw_hbm, mask_hbm,
          tbl, sidx_a, didx_a, sidx_b, didx_b,
          vxi_a, vyi_a, vxi_b, vyi_b,
          o_main, o_sw, o_mask, o_vz,
          sem_ia, sem_ib, sem_va, sem_vb, sem_o):
        wid = lax.axis_index("s") * NC + lax.axis_index("c")
        base0 = wid * e_per_w
        comp_hbms = (xs_hbm, ys_hbm, zs_hbm)

        def start_idx(j, iset, sem):
            base = base0 + j * chunk
            pltpu.async_copy(src_hbm.at[pl.ds(base, chunk)], iset[0], sem)
            pltpu.async_copy(dst_hbm.at[pl.ds(base, chunk)], iset[1], sem)

        def wait_idx(iset, sem):
            for b in iset:
                pltpu.make_async_copy(src_hbm.at[pl.ds(0, chunk)], b,
                                      sem).wait()

        def start_vin(j, vset, sem):
            base = base0 + j * chunk
            pltpu.async_copy(vx_hbm.at[pl.ds(base, chunk)], vset[0], sem)
            pltpu.async_copy(vy_hbm.at[pl.ds(base, chunk)], vset[1], sem)

        def wait_vin(vset, sem):
            for b in vset:
                pltpu.make_async_copy(vx_hbm.at[pl.ds(0, chunk)], b,
                                      sem).wait()

        def diff_pass(comp, final):
            pltpu.sync_copy(comp_hbms[comp], tbl)
            start_idx(0, (sidx_a, didx_a), sem_ia)
            if final:
                start_vin(0, (vxi_a, vyi_a), sem_va)

            def compute(iset, vset, j):
                sidx, didx = iset
                base = base0 + j * chunk

                if final:
                    @plsc.parallel_loop(0, chunk, step=L, unroll=4)
                    def grp(off):
                        sl = pl.ds(off, L)
                        sv = plsc.load_gather(tbl, [sidx[sl]])
                        tv = plsc.load_gather(tbl, [didx[sl]])
                        vz = tv - sv
                        vx = vset[0][sl]
                        vy = vset[1][sl]
                        d2 = vx * vx + vy * vy + vz * vz
                        d2 = jnp.maximum(d2, 1e-12)
                        i = plsc.bitcast(d2, jnp.int32)
                        i = jnp.int32(0x5F3759DF) - (i >> 1)
                        y = plsc.bitcast(i, jnp.float32)
                        for _n in range(2):
                            y = y * (1.5 - 0.5 * d2 * y * y)
                        r = d2 * y
                        mask_b = d2 < _CUT2_EDGE
                        s = jnp.minimum(d2 * inv_cut2, 1.0)
                        q = jnp.full((L,), _SW_C[5], jnp.float32)
                        for c in (_SW_C[4], _SW_C[3], _SW_C[2],
                                  _SW_C[1], _SW_C[0]):
                            q = q * s + c
                        o_vz[sl] = vz
                        o_main[sl] = r
                        o_sw[sl] = jnp.where(mask_b, q, 0.0)
                        o_mask[sl] = jnp.where(mask_b, 1, 0).astype(jnp.int32)

                    pltpu.async_copy(o_vz, vz_hbm.at[pl.ds(base, chunk)],
                                     sem_o)
                    pltpu.async_copy(o_main, dist_hbm.at[pl.ds(base, chunk)],
                                     sem_o)
                    pltpu.async_copy(o_sw, sw_hbm.at[pl.ds(base, chunk)],
                                     sem_o)
                    pltpu.async_copy(o_mask, mask_hbm.at[pl.ds(base, chunk)],
                                     sem_o)
                else:
                    obuf = vset[2]

                    @plsc.parallel_loop(0, chunk, step=L, unroll=4)
                    def grp(off):
                        sl = pl.ds(off, L)
                        sv = plsc.load_gather(tbl, [sidx[sl]])
                        tv = plsc.load_gather(tbl, [didx[sl]])
                        obuf[sl] = tv - sv

                    out_hbm = vx_hbm if comp == 0 else vy_hbm
                    pltpu.async_copy(obuf, out_hbm.at[pl.ds(base, chunk)],
                                     sem_o)

            def drain_outs(j, obuf):
                base = base0 + j * chunk
                if final:
                    for buf, hbm in ((o_vz, vz_hbm), (o_main, dist_hbm),
                                     (o_sw, sw_hbm), (o_mask, mask_hbm)):
                        pltpu.make_async_copy(
                            buf, hbm.at[pl.ds(base, chunk)], sem_o).wait()
                else:
                    hbm = vx_hbm if comp == 0 else vy_hbm
                    pltpu.make_async_copy(
                        obuf, hbm.at[pl.ds(base, chunk)], sem_o).wait()

            def pair_body(p, _):
                j0 = 2 * p
                j1 = j0 + 1
                start_idx(j1, (sidx_b, didx_b), sem_ib)
                if final:
                    start_vin(j1, (vxi_b, vyi_b), sem_vb)
                wait_idx((sidx_a, didx_a), sem_ia)
                if final:
                    wait_vin((vxi_a, vyi_a), sem_va)

                if final:
                    @pl.when(p > 0)
                    def _():
                        drain_outs(j0 - 1, o_main)
                else:
                    @pl.when(p > 0)
                    def _():
                        drain_outs(j0 - 2, o_main)
                compute((sidx_a, didx_a), (vxi_a, vyi_a, o_main), j0)

                @pl.when(j0 + 2 < n_chunks)
                def _():
                    start_idx(j0 + 2, (sidx_a, didx_a), sem_ia)
                    if final:
                        start_vin(j0 + 2, (vxi_a, vyi_a), sem_va)
                wait_idx((sidx_b, didx_b), sem_ib)
                if final:
                    wait_vin((vxi_b, vyi_b), sem_vb)
                if final:
                    drain_outs(j0, o_main)
                else:
                    @pl.when(p > 0)
                    def _():
                        drain_outs(j1 - 2, o_vz)
                compute((sidx_b, didx_b), (vxi_b, vyi_b, o_vz), j1)
                return 0

            lax.fori_loop(0, n_chunks // 2, pair_body, 0)
            if final:
                drain_outs(n_chunks - 1, o_main)
            else:
                drain_outs(n_chunks - 2, o_main)
                drain_outs(n_chunks - 1, o_vz)

        diff_pass(0, False)
        diff_pass(1, False)
        diff_pass(2, True)

    return k


def kernel(coordinates, edge_src, edge_dst):
    n_nodes = coordinates.shape[0]
    n_edges = edge_src.shape[0]
    xs = coordinates[:, 0]
    ys = coordinates[:, 1]
    zs = coordinates[:, 2]
    k = _make_kernel(n_nodes, n_edges, chunk=2000)
    vx, vy, vz, dist, sw, mask = k(xs, ys, zs, edge_src, edge_dst)
    vec = jnp.stack([vx, vy, vz], axis=-1)
    return vec, dist, sw, mask.astype(jnp.bool_)
